# Initial kernel scaffold; baseline (speedup 1.0000x reference)
import functools

import jax
import jax.numpy as jnp
from jax import lax
from jax.experimental import pallas as pl
from jax.experimental.pallas import tpu as pltpu
from jax.experimental.pallas import tpu_sc as plsc

B, L, G = 1024, 50, 26
V, D = 1000000, 32
NC, NS = 2, 16
NW = NC * NS
N_POS = B * L
POS_PER_W = N_POS // NW
POS_CHUNK = 64
ROWS_CHUNK = POS_CHUNK * G
N_CHUNKS = POS_PER_W // POS_CHUNK
N_ROWS = N_POS * G


def _sc_kernel(table_hbm, idx_hbm, pad_hbm, out_hbm,
               idx_v, rows_v, pad_v, pad_s, sem):
    wid = lax.axis_index("s") * NC + lax.axis_index("c")
    pos_base = wid * POS_PER_W
    row_base = pos_base * G

    zeros16 = jnp.zeros((16,), jnp.float32)

    def chunk_body(c, carry):
        rb = row_base + c * ROWS_CHUNK
        pb = pos_base + c * POS_CHUNK
        pltpu.sync_copy(idx_hbm.at[pl.ds(rb, ROWS_CHUNK)], idx_v)
        pltpu.sync_copy(pad_hbm.at[pl.ds(pb, POS_CHUNK)], pad_v)
        pltpu.sync_copy(pad_v, pad_s)
        pltpu.async_copy(table_hbm.at[idx_v], rows_v, sem).wait()

        rows_flat = rows_v.reshape((ROWS_CHUNK * D,))

        def pos_body(p, carry2):
            @pl.when(pad_s[p] != 0)
            def _():
                base = p * (G * D)
                for j in range(G * D // 16):
                    rows_flat[pl.ds(base + j * 16, 16)] = zeros16
            return carry2

        lax.fori_loop(0, POS_CHUNK, pos_body, 0)

        pltpu.sync_copy(rows_v, out_hbm.at[pl.ds(rb, ROWS_CHUNK)])
        return carry

    lax.fori_loop(0, N_CHUNKS, chunk_body, 0)


@jax.jit
def kernel(feat_matrix, padding, embed_table):
    idx_flat = feat_matrix.reshape((N_ROWS,))
    pad_i32 = padding.reshape((N_POS,)).astype(jnp.int32)

    mesh = plsc.VectorSubcoreMesh(core_axis_name="c", subcore_axis_name="s",
                                  num_cores=NC, num_subcores=NS)
    out = pl.kernel(
        _sc_kernel,
        out_type=jax.ShapeDtypeStruct((N_ROWS, D), jnp.float32),
        mesh=mesh,
        scratch_types=[
            pltpu.VMEM((ROWS_CHUNK,), jnp.int32),
            pltpu.VMEM((ROWS_CHUNK, D), jnp.float32),
            pltpu.VMEM((POS_CHUNK,), jnp.int32),
            pltpu.SMEM((POS_CHUNK,), jnp.int32),
            pltpu.SemaphoreType.DMA,
        ],
    )(embed_table, idx_flat, pad_i32)
    return out.reshape((B, L, G * D))


# speedup vs baseline: 6.1251x; 6.1251x over previous
import functools

import jax
import jax.numpy as jnp
from jax import lax
from jax.experimental import pallas as pl
from jax.experimental.pallas import tpu as pltpu
from jax.experimental.pallas import tpu_sc as plsc

B, L, G = 1024, 50, 26
V, D = 1000000, 32
NC, NS = 2, 16
NW = NC * NS
N_POS = B * L
POS_PER_W = N_POS // NW
POS_CHUNK = 64
ROWS_CHUNK = POS_CHUNK * G
N_CHUNKS = POS_PER_W // POS_CHUNK
N_ROWS = N_POS * G


def _sc_kernel(table_hbm, idx_hbm, pad_hbm, out_hbm,
               idx_v, rows_v, pad_v, scale_v, sem):
    wid = lax.axis_index("s") * NC + lax.axis_index("c")
    pos_base = wid * POS_PER_W
    row_base = pos_base * G

    def chunk_body(c, carry):
        rb = row_base + c * ROWS_CHUNK
        pb = pos_base + c * POS_CHUNK
        pltpu.sync_copy(idx_hbm.at[pl.ds(rb, ROWS_CHUNK)], idx_v)
        pltpu.sync_copy(pad_hbm.at[pl.ds(pb, POS_CHUNK)], pad_v)
        pltpu.async_copy(table_hbm.at[idx_v], rows_v, sem).wait()

        for q in range(POS_CHUNK // 16):
            pq = pad_v[pl.ds(q * 16, 16)]
            scale_v[pl.ds(q * 16, 16)] = 1.0 - pq.astype(jnp.float32)

        def pos_body(p, carry2):
            splat = plsc.load_gather(scale_v, [jnp.full((16,), p, jnp.int32)])
            base = p * G
            for r in range(G):
                rows_v[base + r, pl.ds(0, 16)] = (
                    rows_v[base + r, pl.ds(0, 16)] * splat)
                rows_v[base + r, pl.ds(16, 16)] = (
                    rows_v[base + r, pl.ds(16, 16)] * splat)
            return carry2

        lax.fori_loop(0, POS_CHUNK, pos_body, 0)

        pltpu.sync_copy(rows_v, out_hbm.at[pl.ds(rb, ROWS_CHUNK)])
        return carry

    lax.fori_loop(0, N_CHUNKS, chunk_body, 0)


@jax.jit
def kernel(feat_matrix, padding, embed_table):
    idx_flat = feat_matrix.reshape((N_ROWS,))
    pad_i32 = padding.reshape((N_POS,)).astype(jnp.int32)

    mesh = plsc.VectorSubcoreMesh(core_axis_name="c", subcore_axis_name="s",
                                  num_cores=NC, num_subcores=NS)
    out = pl.kernel(
        _sc_kernel,
        out_type=jax.ShapeDtypeStruct((N_ROWS, D), jnp.float32),
        mesh=mesh,
        scratch_types=[
            pltpu.VMEM((ROWS_CHUNK,), jnp.int32),
            pltpu.VMEM((ROWS_CHUNK, D), jnp.float32),
            pltpu.VMEM((POS_CHUNK,), jnp.int32),
            pltpu.VMEM((POS_CHUNK,), jnp.float32),
            pltpu.SemaphoreType.DMA,
        ],
        compiler_params=pltpu.CompilerParams(use_tc_tiling_on_sc=False,
                                             needs_layout_passes=False),
    )(embed_table, idx_flat, pad_i32)
    return out.reshape((B, L, G * D))


# speedup vs baseline: 6.7229x; 1.0976x over previous
import functools

import jax
import jax.numpy as jnp
from jax import lax
from jax.experimental import pallas as pl
from jax.experimental.pallas import tpu as pltpu
from jax.experimental.pallas import tpu_sc as plsc

B, L, G = 1024, 50, 26
V, D = 1000000, 32
NC, NS = 2, 16
NW = NC * NS
N_POS = B * L
POS_PER_W = N_POS // NW
POS_CHUNK = 40
ROWS_CHUNK = POS_CHUNK * G
N_CHUNKS = POS_PER_W // POS_CHUNK
ROWS_PER_W = POS_PER_W * G
N_ROWS = N_POS * G


def _sc_kernel(table_hbm, idx_hbm, pad_hbm, out_hbm,
               idx_v, pad_v, scale_v, rows0, rows1,
               gsem0, gsem1, osem0, osem1):
    rows = (rows0, rows1)
    gsem = (gsem0, gsem1)
    osem = (osem0, osem1)

    wid = lax.axis_index("s") * NC + lax.axis_index("c")
    pos_base = wid * POS_PER_W
    row_base = pos_base * G

    pltpu.sync_copy(idx_hbm.at[pl.ds(row_base, ROWS_PER_W)], idx_v)
    pltpu.sync_copy(pad_hbm.at[pl.ds(pos_base, POS_PER_W)], pad_v)

    def scale_body(q, carry):
        off = pl.multiple_of(q * 16, 16)
        pq = pad_v[pl.ds(off, 16)]
        scale_v[pl.ds(off, 16)] = 1.0 - pq.astype(jnp.float32)
        return carry

    lax.fori_loop(0, POS_PER_W // 16, scale_body, 0)

    def gather_start(c, b):
        off = pl.multiple_of(c * ROWS_CHUNK, 8)
        pltpu.async_copy(table_hbm.at[idx_v.at[pl.ds(off, ROWS_CHUNK)]],
                         rows[b], gsem[b])

    def gather_wait(b):
        pltpu.make_async_copy(out_hbm.at[pl.ds(0, ROWS_CHUNK)],
                              rows[b], gsem[b]).wait()

    def out_start(c, b):
        off = pl.multiple_of(row_base + c * ROWS_CHUNK, 8)
        pltpu.async_copy(rows[b], out_hbm.at[pl.ds(off, ROWS_CHUNK)], osem[b])

    def out_wait(b):
        pltpu.make_async_copy(rows[b], out_hbm.at[pl.ds(0, ROWS_CHUNK)],
                              osem[b]).wait()

    def mask_chunk(c, b):
        def pos_body(p, carry):
            gp = c * POS_CHUNK + p
            splat = plsc.load_gather(scale_v, [jnp.full((16,), gp, jnp.int32)])
            base = p * G
            for r in range(G):
                rows[b][base + r, pl.ds(0, 16)] = (
                    rows[b][base + r, pl.ds(0, 16)] * splat)
                rows[b][base + r, pl.ds(16, 16)] = (
                    rows[b][base + r, pl.ds(16, 16)] * splat)
            return carry

        lax.fori_loop(0, POS_CHUNK, pos_body, 0)

    gather_start(0, 0)

    def group_body(g, carry):
        for b in (0, 1):
            c = g * 2 + b

            @pl.when(c + 1 < N_CHUNKS)
            def _():
                @pl.when(c >= 1)
                def _():
                    out_wait(1 - b)
                gather_start(c + 1, 1 - b)

            gather_wait(b)
            mask_chunk(c, b)
            out_start(c, b)
        return carry

    lax.fori_loop(0, N_CHUNKS // 2, group_body, 0)
    out_wait(0)
    out_wait(1)


@jax.jit
def kernel(feat_matrix, padding, embed_table):
    idx_flat = feat_matrix.reshape((N_ROWS,))
    pad_i32 = padding.reshape((N_POS,)).astype(jnp.int32)

    mesh = plsc.VectorSubcoreMesh(core_axis_name="c", subcore_axis_name="s",
                                  num_cores=NC, num_subcores=NS)
    out = pl.kernel(
        _sc_kernel,
        out_type=jax.ShapeDtypeStruct((N_ROWS, D), jnp.float32),
        mesh=mesh,
        scratch_types=[
            pltpu.VMEM((ROWS_PER_W,), jnp.int32),
            pltpu.VMEM((POS_PER_W,), jnp.int32),
            pltpu.VMEM((POS_PER_W,), jnp.float32),
            pltpu.VMEM((ROWS_CHUNK, D), jnp.float32),
            pltpu.VMEM((ROWS_CHUNK, D), jnp.float32),
            pltpu.SemaphoreType.DMA,
            pltpu.SemaphoreType.DMA,
            pltpu.SemaphoreType.DMA,
            pltpu.SemaphoreType.DMA,
        ],
        compiler_params=pltpu.CompilerParams(use_tc_tiling_on_sc=False,
                                             needs_layout_passes=False),
    )(embed_table, idx_flat, pad_i32)
    return out.reshape((B, L, G * D))
